# baseline (speedup 1.0000x reference)
import functools

import jax
import jax.numpy as jnp
from jax import lax
from jax.experimental import pallas as pl
from jax.experimental.pallas import tpu as pltpu
from jax.experimental.pallas import tpu_sc as plsc


def _mm_bias_body(h_ref, w_ref, b_ref, o_ref):
  o_ref[...] = (
      jnp.dot(h_ref[...], w_ref[...], preferred_element_type=jnp.float32)
      + b_ref[...]
  )


def _mm_bias(h, w, b2d, bn):
  n, k = h.shape
  m = w.shape[1]
  return pl.pallas_call(
      _mm_bias_body,
      grid=(n // bn,),
      in_specs=[
          pl.BlockSpec((bn, k), lambda j: (j, 0)),
          pl.BlockSpec((k, m), lambda j: (0, 0)),
          pl.BlockSpec((1, m), lambda j: (0, 0)),
      ],
      out_specs=pl.BlockSpec((bn, m), lambda j: (j, 0)),
      out_shape=jax.ShapeDtypeStruct((n, m), jnp.float32),
  )(h, w, b2d)


def _mm_cat(h, w, b2d, bn):
  n, d = h.shape
  m = w.shape[1]
  nkc = m // 128
  nj = n // bn
  return pl.pallas_call(
      _mm_bias_body,
      grid=(nj, nkc),
      in_specs=[
          pl.BlockSpec((bn, d), lambda j, k: (j, 0)),
          pl.BlockSpec((d, 128), lambda j, k: (0, k)),
          pl.BlockSpec((1, 128), lambda j, k: (0, k)),
      ],
      out_specs=pl.BlockSpec((bn, 128), lambda j, k: (k * nj + j, 0)),
      out_shape=jax.ShapeDtypeStruct((n * nkc, 128), jnp.float32),
  )(h, w, b2d)


def _cmb(r0_ref, r1_ref, agg_ref, h_ref, a_ref):
  root = jnp.concatenate([r0_ref[...], r1_ref[...]], axis=1)
  v = root + agg_ref[...] + h_ref[...]
  a = a_ref[0, 0]
  v = jnp.maximum(v, 0.0) + a * jnp.minimum(v, 0.0)
  nrm = jnp.sqrt(jnp.sum(v * v, axis=1, keepdims=True))
  return v / jnp.maximum(nrm, 1e-12)


def _layer_body(r0_ref, r1_ref, agg_ref, h_ref, a_ref, w_ref, b_ref,
                x2_ref, h_ref_out):
  hnew = _cmb(r0_ref, r1_ref, agg_ref, h_ref, a_ref)

  @pl.when(pl.program_id(1) == 0)
  def _():
    h_ref_out[...] = hnew

  x2_ref[...] = (
      jnp.dot(hnew, w_ref[...], preferred_element_type=jnp.float32)
      + b_ref[...]
  )


def _layer(x2_prev, agg, h, a2d, w, b2d, bn):
  n, d = h.shape
  m = w.shape[1]
  nkc = m // 128
  nj = n // bn
  return pl.pallas_call(
      _layer_body,
      grid=(nj, nkc),
      in_specs=[
          pl.BlockSpec((bn, 128), lambda j, k: (j, 0)),
          pl.BlockSpec((bn, 128), lambda j, k: (nj + j, 0)),
          pl.BlockSpec((bn, d), lambda j, k: (j, 0)),
          pl.BlockSpec((bn, d), lambda j, k: (j, 0)),
          pl.BlockSpec((1, 1), lambda j, k: (0, 0), memory_space=pltpu.SMEM),
          pl.BlockSpec((d, 128), lambda j, k: (0, k)),
          pl.BlockSpec((1, 128), lambda j, k: (0, k)),
      ],
      out_specs=[
          pl.BlockSpec((bn, 128), lambda j, k: (k * nj + j, 0)),
          pl.BlockSpec((bn, d), lambda j, k: (j, 0)),
      ],
      out_shape=[
          jax.ShapeDtypeStruct((n * nkc, 128), jnp.float32),
          jax.ShapeDtypeStruct((n, d), jnp.float32),
      ],
  )(x2_prev, x2_prev, agg, h, a2d, w, b2d)


def _tail_body(r0_ref, r1_ref, agg_ref, h_ref, a_ref,
               gw1_ref, gb1_ref, gw2_ref, gb2_ref, batch_ref,
               fw1_ref, fb1_ref, fw2_ref, fb2_ref, w3_ref, b3_ref,
               o_ref, acc_ref):
  j = pl.program_id(0)
  hnew = _cmb(r0_ref, r1_ref, agg_ref, h_ref, a_ref)
  t = jnp.maximum(
      jnp.dot(hnew, gw1_ref[...], preferred_element_type=jnp.float32)
      + gb1_ref[...], 0.0)
  t = jnp.dot(t, gw2_ref[...], preferred_element_type=jnp.float32) + gb2_ref[...]
  bn, g = h_ref.shape[0], acc_ref.shape[0]
  onehot = (batch_ref[...] ==
            lax.broadcasted_iota(jnp.int32, (bn, g), 1)).astype(jnp.float32)
  part = lax.dot_general(onehot, t, (((0,), (0,)), ((), ())),
                         preferred_element_type=jnp.float32)

  @pl.when(j == 0)
  def _():
    acc_ref[...] = part

  @pl.when(j > 0)
  def _():
    acc_ref[...] = acc_ref[...] + part

  @pl.when(j == pl.num_programs(0) - 1)
  def _():
    a = jnp.maximum(
        jnp.dot(acc_ref[...], fw1_ref[...], preferred_element_type=jnp.float32)
        + fb1_ref[...], 0.0)
    bb = jnp.maximum(
        jnp.dot(a, fw2_ref[...], preferred_element_type=jnp.float32)
        + fb2_ref[...], 0.0)
    o_ref[...] = jnp.maximum(
        jnp.dot(bb, w3_ref[...], preferred_element_type=jnp.float32)
        + b3_ref[...], 0.0)


def _tail(x2, agg, h, a2d, gw1, gb1, gw2, gb2, batch2d,
          fw1, fb1, fw2, fb2, w3p, b3p, g, bn):
  n, d = h.shape
  nj = n // bn
  cst = lambda j: (0, 0)
  return pl.pallas_call(
      _tail_body,
      grid=(nj,),
      in_specs=[
          pl.BlockSpec((bn, 128), lambda j: (j, 0)),
          pl.BlockSpec((bn, 128), lambda j: (nj + j, 0)),
          pl.BlockSpec((bn, d), lambda j: (j, 0)),
          pl.BlockSpec((bn, d), lambda j: (j, 0)),
          pl.BlockSpec((1, 1), cst, memory_space=pltpu.SMEM),
          pl.BlockSpec((d, d), cst),
          pl.BlockSpec((1, d), cst),
          pl.BlockSpec((d, d), cst),
          pl.BlockSpec((1, d), cst),
          pl.BlockSpec((bn, 1), lambda j: (j, 0)),
          pl.BlockSpec((d, 1024), cst),
          pl.BlockSpec((1, 1024), cst),
          pl.BlockSpec((1024, 512), cst),
          pl.BlockSpec((1, 512), cst),
          pl.BlockSpec((512, 128), cst),
          pl.BlockSpec((1, 128), cst),
      ],
      out_specs=pl.BlockSpec((g, 128), cst),
      out_shape=jax.ShapeDtypeStruct((g, 128), jnp.float32),
      scratch_shapes=[pltpu.VMEM((g, d), jnp.float32)],
  )(x2, x2, agg, h, a2d, gw1, gb1, gw2, gb2, batch2d,
    fw1, fb1, fw2, fb2, w3p, b3p)


_SC_NSUB = 16
_SC_CHUNK = 128
_GBITS = 18


def _make_sc_agg(n, n_chunks, acc_rows):
  zrows = acc_rows // _SC_NSUB
  tail = n - (_SC_NSUB - 1) * zrows
  assert zrows % 64 == 0 and tail % 8 == 0 and 0 < tail <= zrows
  mesh = plsc.VectorSubcoreMesh(core_axis_name="c", subcore_axis_name="s")

  @functools.partial(
      pl.kernel,
      out_type=jax.ShapeDtypeStruct((n, 256), jnp.float32),
      mesh=mesh,
      scratch_types=dict(
          acc=pltpu.VMEM_SHARED((acc_rows, 128), jnp.float32),
          packed_v=pltpu.VMEM((n_chunks, _SC_CHUNK), jnp.uint32),
          gbuf=pltpu.VMEM((2, _SC_CHUNK), jnp.int32),
          dbuf=pltpu.VMEM((2, _SC_CHUNK), jnp.int32),
          buf0=pltpu.VMEM((_SC_CHUNK, 128), jnp.float32),
          buf1=pltpu.VMEM((_SC_CHUNK, 128), jnp.float32),
          sem0=pltpu.SemaphoreType.DMA,
          sem1=pltpu.SemaphoreType.DMA,
      ),
  )
  def agg_kernel(x2_hbm, packed_hbm, out_hbm, acc, packed_v, gbuf, dbuf,
                 buf0, buf1, sem0, sem1):
    c = lax.axis_index("c")
    s = lax.axis_index("s")
    goff = c * n
    gmask = jnp.uint32((1 << _GBITS) - 1)

    zv = jnp.zeros((16,), jnp.float32)

    def zrow(i, _):
      for jj in range(8):
        buf0[i, pl.ds(jj * 16, 16)] = zv
      return 0

    lax.fori_loop(0, 64, zrow, 0)

    for b in range(zrows // 64):
      pltpu.sync_copy(buf0.at[pl.ds(0, 64)],
                      acc.at[pl.ds(s * zrows + b * 64, 64)])

    pltpu.sync_copy(packed_hbm.at[s], packed_v)

    plsc.subcore_barrier()

    def unpack(j, p):
      for k in range(_SC_CHUNK // 16):
        v = packed_v[j, pl.ds(k * 16, 16)]
        gbuf[p, pl.ds(k * 16, 16)] = (v & gmask).astype(jnp.int32) + goff
        dbuf[p, pl.ds(k * 16, 16)] = (v >> _GBITS).astype(jnp.int32)

    unpack(0, 0)
    pltpu.async_copy(x2_hbm.at[gbuf.at[0]], buf0, sem0)

    def chunk2(j0, _):
      unpack(j0 + 1, 1)
      pltpu.async_copy(x2_hbm.at[gbuf.at[1]], buf1, sem1)
      pltpu.make_async_copy(x2_hbm.at[gbuf.at[0]], buf0, sem0).wait()
      pltpu.sync_copy(buf0, acc.at[dbuf.at[0]], add=True)

      @pl.when(j0 + 2 < n_chunks)
      def _():
        unpack(j0 + 2, 0)
        pltpu.async_copy(x2_hbm.at[gbuf.at[0]], buf0, sem0)

      pltpu.make_async_copy(x2_hbm.at[gbuf.at[1]], buf1, sem1).wait()
      pltpu.sync_copy(buf1, acc.at[dbuf.at[1]], add=True)
      return 0

    lax.fori_loop(0, n_chunks // 2, lambda t, u: chunk2(t * 2, u), 0)

    plsc.subcore_barrier()

    @pl.when(s < _SC_NSUB - 1)
    def _():
      pltpu.sync_copy(
          acc.at[pl.ds(s * zrows, zrows)],
          out_hbm.at[pl.ds(s * zrows, zrows), pl.ds(c * 128, 128)])

    @pl.when(s == _SC_NSUB - 1)
    def _():
      pltpu.sync_copy(
          acc.at[pl.ds((_SC_NSUB - 1) * zrows, tail)],
          out_hbm.at[pl.ds((_SC_NSUB - 1) * zrows, tail), pl.ds(c * 128, 128)])

  return agg_kernel


def kernel(x, edge_index, edge_type, batch, enc_W, enc_b, prelu_a,
           conv_Wrel, conv_Wroot, conv_b,
           gp_W1, gp_b1, gp_W2, gp_b2,
           fc_W1, fc_b1, fc_W2, fc_b2, out_W, out_b):
  n, f_in = x.shape
  e = edge_type.shape[0]
  l_layers, r_rel, d, _ = conv_Wrel.shape
  g = 32
  bn = 2000

  src = edge_index[0]
  dst = edge_index[1]

  per_sub = e // _SC_NSUB
  n_chunks = -(-per_sub // _SC_CHUNK)
  n_chunks += n_chunks % 2
  pad = n_chunks * _SC_CHUNK - per_sub
  acc_rows = ((n + _SC_NSUB * 128 - 1) // (_SC_NSUB * 128)) * (_SC_NSUB * 128)

  gbase = (2 + 2 * edge_type) * n + src
  gbase = gbase.reshape(_SC_NSUB, per_sub)
  gbase = jnp.pad(gbase, ((0, 0), (0, pad)))
  dstp = dst.reshape(_SC_NSUB, per_sub)
  dstp = jnp.pad(dstp, ((0, 0), (0, pad)), constant_values=n)
  packed = gbase.astype(jnp.uint32) | (dstp.astype(jnp.uint32) << _GBITS)
  packed = packed.reshape(_SC_NSUB, n_chunks, _SC_CHUNK)

  wcat = jnp.concatenate(
      [conv_Wroot,
       conv_Wrel.transpose(0, 2, 1, 3).reshape(l_layers, d, r_rel * d)],
      axis=2)
  bcat = jnp.concatenate(
      [conv_b, jnp.zeros((l_layers, r_rel * d), jnp.float32)], axis=1)

  a2d = jnp.full((1, 1), prelu_a, jnp.float32)
  batch2d = batch.reshape(n, 1)

  sc_agg = _make_sc_agg(n, n_chunks, acc_rows)

  h = _mm_bias(x, enc_W, enc_b.reshape(1, -1), bn)
  x2 = _mm_cat(h, wcat[0], bcat[0].reshape(1, -1), bn)

  for i in range(1, l_layers):
    agg = sc_agg(x2, packed)
    x2, h = _layer(x2, agg, h, a2d, wcat[i], bcat[i].reshape(1, -1), bn)

  agg = sc_agg(x2, packed)

  w3p = jnp.pad(out_W, ((0, 0), (0, 127)))
  b3p = jnp.pad(out_b, (0, 127)).reshape(1, 128)
  out = _tail(x2, agg, h, a2d,
              gp_W1, gp_b1.reshape(1, -1), gp_W2, gp_b2.reshape(1, -1),
              batch2d, fc_W1, fc_b1.reshape(1, -1), fc_W2, fc_b2.reshape(1, -1),
              w3p, b3p, g, bn)
  return out[:, :1]


# speedup vs baseline: 1.0880x; 1.0880x over previous
import functools

import jax
import jax.numpy as jnp
from jax import lax
from jax.experimental import pallas as pl
from jax.experimental.pallas import tpu as pltpu
from jax.experimental.pallas import tpu_sc as plsc


def _mm_bias_body(h_ref, w_ref, b_ref, o_ref):
  o_ref[...] = (
      jnp.dot(h_ref[...], w_ref[...], preferred_element_type=jnp.float32)
      + b_ref[...]
  )


def _mm_bias(h, w, b2d, bn):
  n, k = h.shape
  m = w.shape[1]
  return pl.pallas_call(
      _mm_bias_body,
      grid=(n // bn,),
      in_specs=[
          pl.BlockSpec((bn, k), lambda j: (j, 0)),
          pl.BlockSpec((k, m), lambda j: (0, 0)),
          pl.BlockSpec((1, m), lambda j: (0, 0)),
      ],
      out_specs=pl.BlockSpec((bn, m), lambda j: (j, 0)),
      out_shape=jax.ShapeDtypeStruct((n, m), jnp.float32),
  )(h, w, b2d)


def _mm_cat(h, w, b2d, bn):
  n, d = h.shape
  m = w.shape[1]
  nkc = m // 128
  nj = n // bn
  return pl.pallas_call(
      _mm_bias_body,
      grid=(nj, nkc),
      in_specs=[
          pl.BlockSpec((bn, d), lambda j, k: (j, 0)),
          pl.BlockSpec((d, 128), lambda j, k: (0, k)),
          pl.BlockSpec((1, 128), lambda j, k: (0, k)),
      ],
      out_specs=pl.BlockSpec((bn, 128), lambda j, k: (k * nj + j, 0)),
      out_shape=jax.ShapeDtypeStruct((n * nkc, 128), jnp.float32),
  )(h, w, b2d)


def _cmb(r0_ref, r1_ref, agg_ref, h_ref, a_ref):
  root = jnp.concatenate([r0_ref[...], r1_ref[...]], axis=1)
  v = root + agg_ref[...] + h_ref[...]
  a = a_ref[0, 0]
  v = jnp.maximum(v, 0.0) + a * jnp.minimum(v, 0.0)
  nrm = jnp.sqrt(jnp.sum(v * v, axis=1, keepdims=True))
  return v / jnp.maximum(nrm, 1e-12)


def _layer_body(r0_ref, r1_ref, agg_ref, h_ref, a_ref, w_ref, b_ref,
                x2_ref, h_ref_out):
  @pl.when(pl.program_id(1) == 0)
  def _():
    h_ref_out[...] = _cmb(r0_ref, r1_ref, agg_ref, h_ref, a_ref)

  x2_ref[...] = (
      jnp.dot(h_ref_out[...], w_ref[...], preferred_element_type=jnp.float32)
      + b_ref[...]
  )


def _layer(x2_prev, agg, h, a2d, w, b2d, bn):
  n, d = h.shape
  m = w.shape[1]
  nkc = m // 128
  nj = n // bn
  return pl.pallas_call(
      _layer_body,
      grid=(nj, nkc),
      in_specs=[
          pl.BlockSpec((bn, 128), lambda j, k: (j, 0)),
          pl.BlockSpec((bn, 128), lambda j, k: (nj + j, 0)),
          pl.BlockSpec((bn, d), lambda j, k: (j, 0)),
          pl.BlockSpec((bn, d), lambda j, k: (j, 0)),
          pl.BlockSpec((1, 1), lambda j, k: (0, 0), memory_space=pltpu.SMEM),
          pl.BlockSpec((d, 128), lambda j, k: (0, k)),
          pl.BlockSpec((1, 128), lambda j, k: (0, k)),
      ],
      out_specs=[
          pl.BlockSpec((bn, 128), lambda j, k: (k * nj + j, 0)),
          pl.BlockSpec((bn, d), lambda j, k: (j, 0)),
      ],
      out_shape=[
          jax.ShapeDtypeStruct((n * nkc, 128), jnp.float32),
          jax.ShapeDtypeStruct((n, d), jnp.float32),
      ],
  )(x2_prev, x2_prev, agg, h, a2d, w, b2d)


def _tail_body(r0_ref, r1_ref, agg_ref, h_ref, a_ref,
               gw1_ref, gb1_ref, gw2_ref, gb2_ref, batch_ref,
               fw1_ref, fb1_ref, fw2_ref, fb2_ref, w3_ref, b3_ref,
               o_ref, acc_ref):
  j = pl.program_id(0)
  hnew = _cmb(r0_ref, r1_ref, agg_ref, h_ref, a_ref)
  t = jnp.maximum(
      jnp.dot(hnew, gw1_ref[...], preferred_element_type=jnp.float32)
      + gb1_ref[...], 0.0)
  t = jnp.dot(t, gw2_ref[...], preferred_element_type=jnp.float32) + gb2_ref[...]
  bn, g = h_ref.shape[0], acc_ref.shape[0]
  onehot = (batch_ref[...] ==
            lax.broadcasted_iota(jnp.int32, (bn, g), 1)).astype(jnp.float32)
  part = lax.dot_general(onehot, t, (((0,), (0,)), ((), ())),
                         preferred_element_type=jnp.float32)

  @pl.when(j == 0)
  def _():
    acc_ref[...] = part

  @pl.when(j > 0)
  def _():
    acc_ref[...] = acc_ref[...] + part

  @pl.when(j == pl.num_programs(0) - 1)
  def _():
    a = jnp.maximum(
        jnp.dot(acc_ref[...], fw1_ref[...], preferred_element_type=jnp.float32)
        + fb1_ref[...], 0.0)
    bb = jnp.maximum(
        jnp.dot(a, fw2_ref[...], preferred_element_type=jnp.float32)
        + fb2_ref[...], 0.0)
    o_ref[...] = jnp.maximum(
        jnp.dot(bb, w3_ref[...], preferred_element_type=jnp.float32)
        + b3_ref[...], 0.0)


def _tail(x2, agg, h, a2d, gw1, gb1, gw2, gb2, batch2d,
          fw1, fb1, fw2, fb2, w3p, b3p, g, bn):
  n, d = h.shape
  nj = n // bn
  cst = lambda j: (0, 0)
  return pl.pallas_call(
      _tail_body,
      grid=(nj,),
      in_specs=[
          pl.BlockSpec((bn, 128), lambda j: (j, 0)),
          pl.BlockSpec((bn, 128), lambda j: (nj + j, 0)),
          pl.BlockSpec((bn, d), lambda j: (j, 0)),
          pl.BlockSpec((bn, d), lambda j: (j, 0)),
          pl.BlockSpec((1, 1), cst, memory_space=pltpu.SMEM),
          pl.BlockSpec((d, d), cst),
          pl.BlockSpec((1, d), cst),
          pl.BlockSpec((d, d), cst),
          pl.BlockSpec((1, d), cst),
          pl.BlockSpec((bn, 1), lambda j: (j, 0)),
          pl.BlockSpec((d, 1024), cst),
          pl.BlockSpec((1, 1024), cst),
          pl.BlockSpec((1024, 512), cst),
          pl.BlockSpec((1, 512), cst),
          pl.BlockSpec((512, 128), cst),
          pl.BlockSpec((1, 128), cst),
      ],
      out_specs=pl.BlockSpec((g, 128), cst),
      out_shape=jax.ShapeDtypeStruct((g, 128), jnp.float32),
      scratch_shapes=[pltpu.VMEM((g, d), jnp.float32)],
  )(x2, x2, agg, h, a2d, gw1, gb1, gw2, gb2, batch2d,
    fw1, fb1, fw2, fb2, w3p, b3p)


_SC_NSUB = 16
_SC_CHUNK = 128
_GBITS = 18


def _make_sc_agg(n, n_chunks, acc_rows):
  zrows = acc_rows // _SC_NSUB
  tail = n - (_SC_NSUB - 1) * zrows
  assert zrows % 64 == 0 and tail % 8 == 0 and 0 < tail <= zrows
  mesh = plsc.VectorSubcoreMesh(core_axis_name="c", subcore_axis_name="s")

  @functools.partial(
      pl.kernel,
      out_type=jax.ShapeDtypeStruct((n, 256), jnp.float32),
      mesh=mesh,
      scratch_types=dict(
          acc=pltpu.VMEM_SHARED((acc_rows, 128), jnp.float32),
          packed_v=pltpu.VMEM((n_chunks, _SC_CHUNK), jnp.uint32),
          gbuf=pltpu.VMEM((2, _SC_CHUNK), jnp.int32),
          dbuf=pltpu.VMEM((2, _SC_CHUNK), jnp.int32),
          buf0=pltpu.VMEM((_SC_CHUNK, 128), jnp.float32),
          buf1=pltpu.VMEM((_SC_CHUNK, 128), jnp.float32),
          sem0=pltpu.SemaphoreType.DMA,
          sem1=pltpu.SemaphoreType.DMA,
      ),
  )
  def agg_kernel(x2_hbm, packed_hbm, out_hbm, acc, packed_v, gbuf, dbuf,
                 buf0, buf1, sem0, sem1):
    c = lax.axis_index("c")
    s = lax.axis_index("s")
    goff = c * n
    gmask = jnp.uint32((1 << _GBITS) - 1)

    zv = jnp.zeros((16,), jnp.float32)

    def zrow(i, _):
      for jj in range(8):
        buf0[i, pl.ds(jj * 16, 16)] = zv
      return 0

    lax.fori_loop(0, 64, zrow, 0)

    for b in range(zrows // 64):
      pltpu.sync_copy(buf0.at[pl.ds(0, 64)],
                      acc.at[pl.ds(s * zrows + b * 64, 64)])

    pltpu.sync_copy(packed_hbm.at[s], packed_v)

    plsc.subcore_barrier()

    def unpack(j, p):
      for k in range(_SC_CHUNK // 16):
        v = packed_v[j, pl.ds(k * 16, 16)]
        gbuf[p, pl.ds(k * 16, 16)] = (v & gmask).astype(jnp.int32) + goff
        dbuf[p, pl.ds(k * 16, 16)] = (v >> _GBITS).astype(jnp.int32)

    unpack(0, 0)
    pltpu.async_copy(x2_hbm.at[gbuf.at[0]], buf0, sem0)

    def chunk2(j0, _):
      unpack(j0 + 1, 1)
      pltpu.async_copy(x2_hbm.at[gbuf.at[1]], buf1, sem1)
      pltpu.make_async_copy(x2_hbm.at[gbuf.at[0]], buf0, sem0).wait()
      pltpu.sync_copy(buf0, acc.at[dbuf.at[0]], add=True)

      @pl.when(j0 + 2 < n_chunks)
      def _():
        unpack(j0 + 2, 0)
        pltpu.async_copy(x2_hbm.at[gbuf.at[0]], buf0, sem0)

      pltpu.make_async_copy(x2_hbm.at[gbuf.at[1]], buf1, sem1).wait()
      pltpu.sync_copy(buf1, acc.at[dbuf.at[1]], add=True)
      return 0

    lax.fori_loop(0, n_chunks // 2, lambda t, u: chunk2(t * 2, u), 0)

    plsc.subcore_barrier()

    @pl.when(s < _SC_NSUB - 1)
    def _():
      pltpu.sync_copy(
          acc.at[pl.ds(s * zrows, zrows)],
          out_hbm.at[pl.ds(s * zrows, zrows), pl.ds(c * 128, 128)])

    @pl.when(s == _SC_NSUB - 1)
    def _():
      pltpu.sync_copy(
          acc.at[pl.ds((_SC_NSUB - 1) * zrows, tail)],
          out_hbm.at[pl.ds((_SC_NSUB - 1) * zrows, tail), pl.ds(c * 128, 128)])

  return agg_kernel


def kernel(x, edge_index, edge_type, batch, enc_W, enc_b, prelu_a,
           conv_Wrel, conv_Wroot, conv_b,
           gp_W1, gp_b1, gp_W2, gp_b2,
           fc_W1, fc_b1, fc_W2, fc_b2, out_W, out_b):
  n, f_in = x.shape
  e = edge_type.shape[0]
  l_layers, r_rel, d, _ = conv_Wrel.shape
  g = 32
  bn = 2000

  src = edge_index[0]
  dst = edge_index[1]

  per_sub = e // _SC_NSUB
  n_chunks = -(-per_sub // _SC_CHUNK)
  n_chunks += n_chunks % 2
  pad = n_chunks * _SC_CHUNK - per_sub
  acc_rows = ((n + _SC_NSUB * 128 - 1) // (_SC_NSUB * 128)) * (_SC_NSUB * 128)

  gbase = (2 + 2 * edge_type) * n + src
  gbase = gbase.reshape(_SC_NSUB, per_sub)
  gbase = jnp.pad(gbase, ((0, 0), (0, pad)))
  dstp = dst.reshape(_SC_NSUB, per_sub)
  dstp = jnp.pad(dstp, ((0, 0), (0, pad)), constant_values=n)
  packed = gbase.astype(jnp.uint32) | (dstp.astype(jnp.uint32) << _GBITS)
  packed = packed.reshape(_SC_NSUB, n_chunks, _SC_CHUNK)

  wcat = jnp.concatenate(
      [conv_Wroot,
       conv_Wrel.transpose(0, 2, 1, 3).reshape(l_layers, d, r_rel * d)],
      axis=2)
  bcat = jnp.concatenate(
      [conv_b, jnp.zeros((l_layers, r_rel * d), jnp.float32)], axis=1)

  a2d = jnp.full((1, 1), prelu_a, jnp.float32)
  batch2d = batch.reshape(n, 1)

  sc_agg = _make_sc_agg(n, n_chunks, acc_rows)

  h = _mm_bias(x, enc_W, enc_b.reshape(1, -1), bn)
  x2 = _mm_cat(h, wcat[0], bcat[0].reshape(1, -1), bn)

  for i in range(1, l_layers):
    agg = sc_agg(x2, packed)
    x2, h = _layer(x2, agg, h, a2d, wcat[i], bcat[i].reshape(1, -1), bn)

  agg = sc_agg(x2, packed)

  w3p = jnp.pad(out_W, ((0, 0), (0, 127)))
  b3p = jnp.pad(out_b, (0, 127)).reshape(1, 128)
  out = _tail(x2, agg, h, a2d,
              gp_W1, gp_b1.reshape(1, -1), gp_W2, gp_b2.reshape(1, -1),
              batch2d, fc_W1, fc_b1.reshape(1, -1), fc_W2, fc_b2.reshape(1, -1),
              w3p, b3p, g, bn)
  return out[:, :1]
